# baseline (speedup 1.0000x reference)
import functools

import jax
import jax.numpy as jnp
from jax import lax
from jax.experimental import pallas as pl
from jax.experimental.pallas import tpu as pltpu
from jax.experimental.pallas import tpu_sc as plsc

_N = 8192
_S = 1024
_K = 64
_B = 4
_R2 = 0.2 * 0.2
_ROWS = 64
_LANES = 128

_NW = 32
_RPW = (_B * _S) // _NW


def _fps_body(xp_ref, yp_ref, zp_ref, out_ref, dist_ref):
    iota3 = (lax.broadcasted_iota(jnp.int32, (_B, _ROWS, _LANES), 1) * _LANES
             + lax.broadcasted_iota(jnp.int32, (_B, _ROWS, _LANES), 2))
    dist_ref[...] = jnp.full((_B, _ROWS, _LANES), 1e10, jnp.float32)
    c0x = xp_ref[:, 0:1, 0:1]
    c0y = yp_ref[:, 0:1, 0:1]
    c0z = zp_ref[:, 0:1, 0:1]

    def step(i, carry):
        cx, cy, cz = carry
        for b in range(_B):
            out_ref[pl.ds(i, 1), pl.ds(b * 4 + 0, 1)] = cx[b].reshape(1, 1)
            out_ref[pl.ds(i, 1), pl.ds(b * 4 + 1, 1)] = cy[b].reshape(1, 1)
            out_ref[pl.ds(i, 1), pl.ds(b * 4 + 2, 1)] = cz[b].reshape(1, 1)
        xs = xp_ref[...]
        ys = yp_ref[...]
        zs = zp_ref[...]
        dx = xs - cx
        dy = ys - cy
        dz = zs - cz
        d = (dx * dx + dz * dz) + dy * dy
        dmin = jnp.minimum(dist_ref[...], d)
        dist_ref[...] = dmin
        mx = jnp.max(dmin, axis=(1, 2), keepdims=True)
        eqm = dmin == mx
        ncx = jnp.sum(jnp.where(eqm, xs, 0.0), axis=(1, 2), keepdims=True)
        ncy = jnp.sum(jnp.where(eqm, ys, 0.0), axis=(1, 2), keepdims=True)
        ncz = jnp.sum(jnp.where(eqm, zs, 0.0), axis=(1, 2), keepdims=True)
        nmax = jnp.sum(jnp.where(eqm, 1, 0), axis=(1, 2), keepdims=True)

        def tie_path(_):
            nf = jnp.min(jnp.where(eqm, iota3, _N), axis=(1, 2), keepdims=True)
            pm = iota3 == nf
            tx = jnp.sum(jnp.where(pm, xs, 0.0), axis=(1, 2), keepdims=True)
            ty = jnp.sum(jnp.where(pm, ys, 0.0), axis=(1, 2), keepdims=True)
            tz = jnp.sum(jnp.where(pm, zs, 0.0), axis=(1, 2), keepdims=True)
            return tx, ty, tz

        return lax.cond(jnp.max(nmax) > 1, tie_path,
                        lambda _: (ncx, ncy, ncz), 0)

    lax.fori_loop(0, _S, step, (c0x, c0y, c0z))


def _ballq_body(xp, yp, zp, cxp, cyp, czp, out,
                xv, yv, zv, cxv, cyv, czv, selv, outv, idxv, sem):
    cidx = lax.axis_index("c")
    sidx = lax.axis_index("s")
    wid = sidx * 2 + cidx
    b = wid // (_NW // _B)
    sub = wid % (_NW // _B)

    pltpu.sync_copy(xp.at[b], xv)
    pltpu.sync_copy(yp.at[b], yv)
    pltpu.sync_copy(zp.at[b], zv)
    pltpu.sync_copy(cxp.at[b], cxv)
    pltpu.sync_copy(cyp.at[b], cyv)
    pltpu.sync_copy(czp.at[b], czv)

    iota = lax.iota(jnp.int32, 16)
    nb = _NW // _B

    for t16 in range(_RPW // 16):
        idxv[pl.ds(t16 * 16, 16)] = (iota + t16 * 16) * nb + sub + b * _S

    def row(t, carry):
        s = t * nb + sub
        ssplat = jnp.full((16,), s, jnp.int32)
        cxj = plsc.load_gather(cxv, [ssplat])
        cyj = plsc.load_gather(cyv, [ssplat])
        czj = plsc.load_gather(czv, [ssplat])

        def cond(st):
            base, cs = st
            return jnp.logical_and(cs[0] < _K, base < _N)

        def body(st):
            base, cs = st
            for k in range(8):
                xs = xv[pl.ds(base + k * 16, 16)]
                ys = yv[pl.ds(base + k * 16, 16)]
                zs = zv[pl.ds(base + k * 16, 16)]
                dx = xs - cxj
                dy = ys - cyj
                dz = zs - czj
                d2 = (dx * dx + dz * dz) + dy * dy
                m = d2 <= _R2
                pref = plsc.cumsum(m.astype(jnp.int32))
                pos = cs + pref - 1
                plsc.store_scatter(selv, [pos], iota + (base + k * 16),
                                   mask=m)
                cs = cs + plsc.all_reduce_population_count(m)
            return base + 128, cs

        _, cse = lax.while_loop(cond, body,
                                (jnp.int32(0), jnp.zeros((16,), jnp.int32)))
        cnt = jnp.minimum(cse[0], _K)
        first = jnp.where(cnt > 0, selv[pl.ds(0, 16)][0], 0)
        tsplat = jnp.full((16,), t, jnp.int32)
        for k in range(_K // 16):
            lane = iota + (k * 16)
            sel = selv[pl.ds(k * 16, 16)]
            sel = jnp.where(lane < cnt, sel, first)
            gx = plsc.load_gather(xv, [sel]) - cxj
            gy = plsc.load_gather(yv, [sel]) - cyj
            gz = plsc.load_gather(zv, [sel]) - czj
            pos0 = lane * 3
            plsc.store_scatter(outv, [tsplat, pos0], gx)
            plsc.store_scatter(outv, [tsplat, pos0 + 1], gy)
            plsc.store_scatter(outv, [tsplat, pos0 + 2], gz)
        return carry

    lax.fori_loop(0, _RPW, row, jnp.int32(0))
    pltpu.async_copy(outv, out.at[idxv], sem).wait()


@functools.cache
def _build_ballq():
    return functools.partial(
        pl.kernel,
        out_type=jax.ShapeDtypeStruct((_B * _S, 256), jnp.float32),
        mesh=plsc.VectorSubcoreMesh(core_axis_name="c", subcore_axis_name="s"),
        compiler_params=pltpu.CompilerParams(needs_layout_passes=False),
        scratch_types=[
            pltpu.VMEM((_N,), jnp.float32),
            pltpu.VMEM((_N,), jnp.float32),
            pltpu.VMEM((_N,), jnp.float32),
            pltpu.VMEM((_S,), jnp.float32),
            pltpu.VMEM((_S,), jnp.float32),
            pltpu.VMEM((_S,), jnp.float32),
            pltpu.VMEM((192,), jnp.int32),
            pltpu.VMEM((_RPW, 256), jnp.float32),
            pltpu.VMEM((_RPW,), jnp.int32),
            pltpu.SemaphoreType.DMA,
        ],
    )(_ballq_body)


def kernel(xyz):
    xp = xyz[:, :, 0]
    yp = xyz[:, :, 1]
    zp = xyz[:, :, 2]

    cent16 = pl.pallas_call(
        _fps_body,
        out_shape=jax.ShapeDtypeStruct((_S, 16), jnp.float32),
        scratch_shapes=[pltpu.VMEM((_B, _ROWS, _LANES), jnp.float32)],
    )(xp.reshape(_B, _ROWS, _LANES),
      yp.reshape(_B, _ROWS, _LANES),
      zp.reshape(_B, _ROWS, _LANES))

    centroids = cent16.reshape(_S, 4, 4)[:, :, :3].transpose(1, 0, 2)

    flat = _build_ballq()(xp, yp, zp,
                          centroids[:, :, 0], centroids[:, :, 1],
                          centroids[:, :, 2])
    new_xyz = flat[:, :_K * 3].reshape(_B, _S, _K, 3)
    return new_xyz, centroids


# speedup vs baseline: 1.1976x; 1.1976x over previous
import functools

import jax
import jax.numpy as jnp
from jax import lax
from jax.experimental import pallas as pl
from jax.experimental.pallas import tpu as pltpu
from jax.experimental.pallas import tpu_sc as plsc

_N = 8192
_S = 1024
_K = 64
_B = 4
_R2 = 0.2 * 0.2
_ROWS = 64
_LANES = 128

_NW = 32
_RPW = (_B * _S) // _NW


def _fps_body(xp_ref, yp_ref, zp_ref, out_ref, dist_ref):
    iota3 = (lax.broadcasted_iota(jnp.int32, (_B, _ROWS, _LANES), 1) * _LANES
             + lax.broadcasted_iota(jnp.int32, (_B, _ROWS, _LANES), 2))
    dist_ref[...] = jnp.full((_B, _ROWS, _LANES), 1e10, jnp.float32)
    c0x = xp_ref[:, 0:1, 0:1]
    c0y = yp_ref[:, 0:1, 0:1]
    c0z = zp_ref[:, 0:1, 0:1]

    def half(i_out, cx, cy, cz, dprev, xs, ys, zs):
        for b in range(_B):
            out_ref[pl.ds(i_out, 1), pl.ds(b * 4 + 0, 1)] = cx[b].reshape(1, 1)
            out_ref[pl.ds(i_out, 1), pl.ds(b * 4 + 1, 1)] = cy[b].reshape(1, 1)
            out_ref[pl.ds(i_out, 1), pl.ds(b * 4 + 2, 1)] = cz[b].reshape(1, 1)
        dx = xs - cx
        dy = ys - cy
        dz = zs - cz
        d = (dx * dx + dz * dz) + dy * dy
        dmin = jnp.minimum(dprev, d)
        mx = jnp.max(dmin, axis=(1, 2), keepdims=True)
        eqm = dmin == mx
        ncx = jnp.sum(jnp.where(eqm, xs, 0.0), axis=(1, 2), keepdims=True)
        ncy = jnp.sum(jnp.where(eqm, ys, 0.0), axis=(1, 2), keepdims=True)
        ncz = jnp.sum(jnp.where(eqm, zs, 0.0), axis=(1, 2), keepdims=True)
        nmax = jnp.sum(jnp.where(eqm, 1, 0), axis=(1, 2), keepdims=True)

        def tie_path(_):
            nf = jnp.min(jnp.where(eqm, iota3, _N), axis=(1, 2), keepdims=True)
            pm = iota3 == nf
            tx = jnp.sum(jnp.where(pm, xs, 0.0), axis=(1, 2), keepdims=True)
            ty = jnp.sum(jnp.where(pm, ys, 0.0), axis=(1, 2), keepdims=True)
            tz = jnp.sum(jnp.where(pm, zs, 0.0), axis=(1, 2), keepdims=True)
            return tx, ty, tz

        c = lax.cond(jnp.max(nmax) > 1, tie_path,
                     lambda _: (ncx, ncy, ncz), 0)
        return c[0], c[1], c[2], dmin

    def step(i, carry):
        cx, cy, cz = carry
        xs = xp_ref[...]
        ys = yp_ref[...]
        zs = zp_ref[...]
        dprev = dist_ref[...]
        cx, cy, cz, dmin = half(2 * i, cx, cy, cz, dprev, xs, ys, zs)
        cx, cy, cz, dmin = half(2 * i + 1, cx, cy, cz, dmin, xs, ys, zs)
        dist_ref[...] = dmin
        return cx, cy, cz

    lax.fori_loop(0, _S // 2, step, (c0x, c0y, c0z))


def _ballq_body(xp, yp, zp, cxp, cyp, czp, out,
                xv, yv, zv, cxv, cyv, czv, selv, outv, idxv, sem):
    cidx = lax.axis_index("c")
    sidx = lax.axis_index("s")
    wid = sidx * 2 + cidx
    b = wid // (_NW // _B)
    sub = wid % (_NW // _B)

    pltpu.sync_copy(xp.at[b], xv)
    pltpu.sync_copy(yp.at[b], yv)
    pltpu.sync_copy(zp.at[b], zv)
    pltpu.sync_copy(cxp.at[b], cxv)
    pltpu.sync_copy(cyp.at[b], cyv)
    pltpu.sync_copy(czp.at[b], czv)

    iota = lax.iota(jnp.int32, 16)
    nb = _NW // _B

    for t16 in range(_RPW // 16):
        idxv[pl.ds(t16 * 16, 16)] = (iota + t16 * 16) * nb + sub + b * _S

    def row(t, carry):
        s = t * nb + sub
        ssplat = jnp.full((16,), s, jnp.int32)
        cxj = plsc.load_gather(cxv, [ssplat])
        cyj = plsc.load_gather(cyv, [ssplat])
        czj = plsc.load_gather(czv, [ssplat])

        def cond(st):
            base, cnt = st
            return jnp.logical_and(cnt < _K, base < _N)

        def body(st):
            base, cnt = st
            c = cnt
            for k in range(8):
                xs = xv[pl.ds(base + k * 16, 16)]
                ys = yv[pl.ds(base + k * 16, 16)]
                zs = zv[pl.ds(base + k * 16, 16)]
                dx = xs - cxj
                dy = ys - cyj
                dz = zs - czj
                d2 = (dx * dx + dz * dz) + dy * dy
                m = d2 <= _R2
                plsc.store_compressed(selv.at[pl.ds(c, 16)],
                                      iota + (base + k * 16), mask=m)
                c = c + plsc.all_reduce_population_count(m)[0]
            return base + 128, c

        _, cnt = lax.while_loop(cond, body, (jnp.int32(0), jnp.int32(0)))
        cnt = jnp.minimum(cnt, _K)
        first = jnp.where(cnt > 0, selv[pl.ds(0, 16)][0], 0)
        tsplat = jnp.full((16,), t, jnp.int32)
        for k in range(_K // 16):
            lane = iota + (k * 16)
            sel = selv[pl.ds(k * 16, 16)]
            sel = jnp.where(lane < cnt, sel, first)
            gx = plsc.load_gather(xv, [sel]) - cxj
            gy = plsc.load_gather(yv, [sel]) - cyj
            gz = plsc.load_gather(zv, [sel]) - czj
            pos0 = lane * 3
            plsc.store_scatter(outv, [tsplat, pos0], gx)
            plsc.store_scatter(outv, [tsplat, pos0 + 1], gy)
            plsc.store_scatter(outv, [tsplat, pos0 + 2], gz)
        return carry

    lax.fori_loop(0, _RPW, row, jnp.int32(0))
    pltpu.async_copy(outv, out.at[idxv], sem).wait()


@functools.cache
def _build_ballq():
    return functools.partial(
        pl.kernel,
        out_type=jax.ShapeDtypeStruct((_B * _S, 256), jnp.float32),
        mesh=plsc.VectorSubcoreMesh(core_axis_name="c", subcore_axis_name="s"),
        compiler_params=pltpu.CompilerParams(needs_layout_passes=False),
        scratch_types=[
            pltpu.VMEM((_N,), jnp.float32),
            pltpu.VMEM((_N,), jnp.float32),
            pltpu.VMEM((_N,), jnp.float32),
            pltpu.VMEM((_S,), jnp.float32),
            pltpu.VMEM((_S,), jnp.float32),
            pltpu.VMEM((_S,), jnp.float32),
            pltpu.VMEM((192,), jnp.int32),
            pltpu.VMEM((_RPW, 256), jnp.float32),
            pltpu.VMEM((_RPW,), jnp.int32),
            pltpu.SemaphoreType.DMA,
        ],
    )(_ballq_body)


def kernel(xyz):
    xp = xyz[:, :, 0]
    yp = xyz[:, :, 1]
    zp = xyz[:, :, 2]

    cent16 = pl.pallas_call(
        _fps_body,
        out_shape=jax.ShapeDtypeStruct((_S, 16), jnp.float32),
        scratch_shapes=[pltpu.VMEM((_B, _ROWS, _LANES), jnp.float32)],
    )(xp.reshape(_B, _ROWS, _LANES),
      yp.reshape(_B, _ROWS, _LANES),
      zp.reshape(_B, _ROWS, _LANES))

    centroids = cent16.reshape(_S, 4, 4)[:, :, :3].transpose(1, 0, 2)

    flat = _build_ballq()(xp, yp, zp,
                          centroids[:, :, 0], centroids[:, :, 1],
                          centroids[:, :, 2])
    new_xyz = flat[:, :_K * 3].reshape(_B, _S, _K, 3)
    return new_xyz, centroids


# speedup vs baseline: 1.2222x; 1.0205x over previous
import functools

import jax
import jax.numpy as jnp
from jax import lax
from jax.experimental import pallas as pl
from jax.experimental.pallas import tpu as pltpu
from jax.experimental.pallas import tpu_sc as plsc

_N = 8192
_S = 1024
_K = 64
_B = 4
_R2 = 0.2 * 0.2
_ROWS = 64
_LANES = 128

_NW = 32
_RPW = (_B * _S) // _NW


def _fps_body(xp_ref, yp_ref, zp_ref, out_ref, dist_ref):
    iota3 = (lax.broadcasted_iota(jnp.int32, (_B, _ROWS, _LANES), 1) * _LANES
             + lax.broadcasted_iota(jnp.int32, (_B, _ROWS, _LANES), 2))
    dist_ref[...] = jnp.full((_B, _ROWS, _LANES), 1e10, jnp.float32)
    c0x = xp_ref[:, 0:1, 0:1]
    c0y = yp_ref[:, 0:1, 0:1]
    c0z = zp_ref[:, 0:1, 0:1]

    def half(i_out, cx, cy, cz, dprev, xs, ys, zs):
        for b in range(_B):
            out_ref[pl.ds(i_out, 1), pl.ds(b * 4 + 0, 1)] = cx[b].reshape(1, 1)
            out_ref[pl.ds(i_out, 1), pl.ds(b * 4 + 1, 1)] = cy[b].reshape(1, 1)
            out_ref[pl.ds(i_out, 1), pl.ds(b * 4 + 2, 1)] = cz[b].reshape(1, 1)
        dx = xs - cx
        dy = ys - cy
        dz = zs - cz
        d = (dx * dx + dz * dz) + dy * dy
        dmin = jnp.minimum(dprev, d)
        mx = jnp.max(dmin, axis=(1, 2), keepdims=True)
        eqm = dmin == mx
        ncx = jnp.sum(jnp.where(eqm, xs, 0.0), axis=(1, 2), keepdims=True)
        ncy = jnp.sum(jnp.where(eqm, ys, 0.0), axis=(1, 2), keepdims=True)
        ncz = jnp.sum(jnp.where(eqm, zs, 0.0), axis=(1, 2), keepdims=True)
        nmax = jnp.sum(jnp.where(eqm, 1, 0), axis=(1, 2), keepdims=True)

        def tie_path(_):
            nf = jnp.min(jnp.where(eqm, iota3, _N), axis=(1, 2), keepdims=True)
            pm = iota3 == nf
            tx = jnp.sum(jnp.where(pm, xs, 0.0), axis=(1, 2), keepdims=True)
            ty = jnp.sum(jnp.where(pm, ys, 0.0), axis=(1, 2), keepdims=True)
            tz = jnp.sum(jnp.where(pm, zs, 0.0), axis=(1, 2), keepdims=True)
            return tx, ty, tz

        c = lax.cond(jnp.max(nmax) > 1, tie_path,
                     lambda _: (ncx, ncy, ncz), 0)
        return c[0], c[1], c[2], dmin

    def step(i, carry):
        cx, cy, cz = carry
        xs = xp_ref[...]
        ys = yp_ref[...]
        zs = zp_ref[...]
        dmin = dist_ref[...]
        for u in range(4):
            cx, cy, cz, dmin = half(4 * i + u, cx, cy, cz, dmin, xs, ys, zs)
        dist_ref[...] = dmin
        return cx, cy, cz

    lax.fori_loop(0, _S // 4, step, (c0x, c0y, c0z))


def _ballq_body(xp, yp, zp, cxp, cyp, czp, out,
                xv, yv, zv, cxv, cyv, czv, selv, outv, idxv, sem):
    cidx = lax.axis_index("c")
    sidx = lax.axis_index("s")
    wid = sidx * 2 + cidx
    b = wid // (_NW // _B)
    sub = wid % (_NW // _B)

    pltpu.sync_copy(xp.at[b], xv)
    pltpu.sync_copy(yp.at[b], yv)
    pltpu.sync_copy(zp.at[b], zv)
    pltpu.sync_copy(cxp.at[b], cxv)
    pltpu.sync_copy(cyp.at[b], cyv)
    pltpu.sync_copy(czp.at[b], czv)

    iota = lax.iota(jnp.int32, 16)
    nb = _NW // _B

    for t16 in range(_RPW // 16):
        idxv[pl.ds(t16 * 16, 16)] = (iota + t16 * 16) * nb + sub + b * _S

    def row(t, carry):
        s = t * nb + sub
        ssplat = jnp.full((16,), s, jnp.int32)
        cxj = plsc.load_gather(cxv, [ssplat])
        cyj = plsc.load_gather(cyv, [ssplat])
        czj = plsc.load_gather(czv, [ssplat])

        def cond(st):
            base, cnt = st
            return jnp.logical_and(cnt < _K, base < _N)

        def body(st):
            base, cnt = st
            c = cnt
            for k in range(8):
                xs = xv[pl.ds(base + k * 16, 16)]
                ys = yv[pl.ds(base + k * 16, 16)]
                zs = zv[pl.ds(base + k * 16, 16)]
                dx = xs - cxj
                dy = ys - cyj
                dz = zs - czj
                d2 = (dx * dx + dz * dz) + dy * dy
                m = d2 <= _R2
                plsc.store_compressed(selv.at[pl.ds(c, 16)],
                                      iota + (base + k * 16), mask=m)
                c = c + plsc.all_reduce_population_count(m)[0]
            return base + 128, c

        _, cnt = lax.while_loop(cond, body, (jnp.int32(0), jnp.int32(0)))
        cnt = jnp.minimum(cnt, _K)
        first = jnp.where(cnt > 0, selv[pl.ds(0, 16)][0], 0)
        tsplat = jnp.full((16,), t, jnp.int32)
        for k in range(_K // 16):
            lane = iota + (k * 16)
            sel = selv[pl.ds(k * 16, 16)]
            sel = jnp.where(lane < cnt, sel, first)
            gx = plsc.load_gather(xv, [sel]) - cxj
            gy = plsc.load_gather(yv, [sel]) - cyj
            gz = plsc.load_gather(zv, [sel]) - czj
            pos0 = lane * 3
            plsc.store_scatter(outv, [tsplat, pos0], gx)
            plsc.store_scatter(outv, [tsplat, pos0 + 1], gy)
            plsc.store_scatter(outv, [tsplat, pos0 + 2], gz)
        return carry

    lax.fori_loop(0, _RPW, row, jnp.int32(0))
    pltpu.async_copy(outv, out.at[idxv], sem).wait()


@functools.cache
def _build_ballq():
    return functools.partial(
        pl.kernel,
        out_type=jax.ShapeDtypeStruct((_B * _S, 256), jnp.float32),
        mesh=plsc.VectorSubcoreMesh(core_axis_name="c", subcore_axis_name="s"),
        compiler_params=pltpu.CompilerParams(needs_layout_passes=False),
        scratch_types=[
            pltpu.VMEM((_N,), jnp.float32),
            pltpu.VMEM((_N,), jnp.float32),
            pltpu.VMEM((_N,), jnp.float32),
            pltpu.VMEM((_S,), jnp.float32),
            pltpu.VMEM((_S,), jnp.float32),
            pltpu.VMEM((_S,), jnp.float32),
            pltpu.VMEM((192,), jnp.int32),
            pltpu.VMEM((_RPW, 256), jnp.float32),
            pltpu.VMEM((_RPW,), jnp.int32),
            pltpu.SemaphoreType.DMA,
        ],
    )(_ballq_body)


def kernel(xyz):
    xp = xyz[:, :, 0]
    yp = xyz[:, :, 1]
    zp = xyz[:, :, 2]

    cent16 = pl.pallas_call(
        _fps_body,
        out_shape=jax.ShapeDtypeStruct((_S, 16), jnp.float32),
        scratch_shapes=[pltpu.VMEM((_B, _ROWS, _LANES), jnp.float32)],
    )(xp.reshape(_B, _ROWS, _LANES),
      yp.reshape(_B, _ROWS, _LANES),
      zp.reshape(_B, _ROWS, _LANES))

    centroids = cent16.reshape(_S, 4, 4)[:, :, :3].transpose(1, 0, 2)

    flat = _build_ballq()(xp, yp, zp,
                          centroids[:, :, 0], centroids[:, :, 1],
                          centroids[:, :, 2])
    new_xyz = flat[:, :_K * 3].reshape(_B, _S, _K, 3)
    return new_xyz, centroids


# speedup vs baseline: 1.7716x; 1.4495x over previous
import functools

import jax
import jax.numpy as jnp
from jax import lax
from jax.experimental import pallas as pl
from jax.experimental.pallas import tpu as pltpu
from jax.experimental.pallas import tpu_sc as plsc

_N = 8192
_S = 1024
_K = 64
_B = 4
_R2 = 0.2 * 0.2
_ROWS = 64
_LANES = 128

_NW = 32
_RPW = (_B * _S) // _NW


def _fps_body(xp_ref, yp_ref, zp_ref, out_ref, dist_ref):
    iota3 = (lax.broadcasted_iota(jnp.int32, (_B, _ROWS, _LANES), 1) * _LANES
             + lax.broadcasted_iota(jnp.int32, (_B, _ROWS, _LANES), 2))
    dist_ref[...] = jnp.full((_B, _ROWS, _LANES), 1e10, jnp.float32)
    c0x = xp_ref[:, 0:1, 0:1]
    c0y = yp_ref[:, 0:1, 0:1]
    c0z = zp_ref[:, 0:1, 0:1]

    def half(i_out, cx, cy, cz, dprev, xs, ys, zs):
        for b in range(_B):
            out_ref[pl.ds(i_out, 1), pl.ds(b * 4 + 0, 1)] = cx[b].reshape(1, 1)
            out_ref[pl.ds(i_out, 1), pl.ds(b * 4 + 1, 1)] = cy[b].reshape(1, 1)
            out_ref[pl.ds(i_out, 1), pl.ds(b * 4 + 2, 1)] = cz[b].reshape(1, 1)
        dx = xs - cx
        dy = ys - cy
        dz = zs - cz
        d = (dx * dx + dz * dz) + dy * dy
        dmin = jnp.minimum(dprev, d)
        mx = jnp.max(dmin, axis=(1, 2), keepdims=True)
        eqm = dmin == mx
        ncx = jnp.sum(jnp.where(eqm, xs, 0.0), axis=(1, 2), keepdims=True)
        ncy = jnp.sum(jnp.where(eqm, ys, 0.0), axis=(1, 2), keepdims=True)
        ncz = jnp.sum(jnp.where(eqm, zs, 0.0), axis=(1, 2), keepdims=True)
        nmax = jnp.sum(jnp.where(eqm, 1, 0), axis=(1, 2), keepdims=True)

        def tie_path(_):
            nf = jnp.min(jnp.where(eqm, iota3, _N), axis=(1, 2), keepdims=True)
            pm = iota3 == nf
            tx = jnp.sum(jnp.where(pm, xs, 0.0), axis=(1, 2), keepdims=True)
            ty = jnp.sum(jnp.where(pm, ys, 0.0), axis=(1, 2), keepdims=True)
            tz = jnp.sum(jnp.where(pm, zs, 0.0), axis=(1, 2), keepdims=True)
            return tx, ty, tz

        c = lax.cond(jnp.max(nmax) > 1, tie_path,
                     lambda _: (ncx, ncy, ncz), 0)
        return c[0], c[1], c[2], dmin

    def step(i, carry):
        cx, cy, cz = carry
        xs = xp_ref[...]
        ys = yp_ref[...]
        zs = zp_ref[...]
        dmin = dist_ref[...]
        for u in range(4):
            cx, cy, cz, dmin = half(4 * i + u, cx, cy, cz, dmin, xs, ys, zs)
        dist_ref[...] = dmin
        return cx, cy, cz

    lax.fori_loop(0, _S // 4, step, (c0x, c0y, c0z))


def _ballq_body(xp, yp, zp, cxp, cyp, czp, out,
                xv, yv, zv, cxv, cyv, czv, selv, outv, idxv, sem):
    cidx = lax.axis_index("c")
    sidx = lax.axis_index("s")
    wid = sidx * 2 + cidx
    b = wid // (_NW // _B)
    sub = wid % (_NW // _B)

    pltpu.sync_copy(xp.at[b], xv)
    pltpu.sync_copy(yp.at[b], yv)
    pltpu.sync_copy(zp.at[b], zv)
    pltpu.sync_copy(cxp.at[b], cxv)
    pltpu.sync_copy(cyp.at[b], cyv)
    pltpu.sync_copy(czp.at[b], czv)

    iota = lax.iota(jnp.int32, 16)
    nb = _NW // _B

    for t16 in range(_RPW // 16):
        idxv[pl.ds(t16 * 16, 16)] = (iota + t16 * 16) * nb + sub + b * _S

    def row(t, carry):
        s = t * nb + sub
        ssplat = jnp.full((16,), s, jnp.int32)
        cxj = plsc.load_gather(cxv, [ssplat])
        cyj = plsc.load_gather(cyv, [ssplat])
        czj = plsc.load_gather(czv, [ssplat])

        def cond(st):
            base, cnt = st
            return jnp.logical_and(cnt < _K, base < _N)

        def body(st):
            base, cnt = st
            ms = []
            pcs = []
            for k in range(8):
                xs = xv[pl.ds(base + k * 16, 16)]
                ys = yv[pl.ds(base + k * 16, 16)]
                zs = zv[pl.ds(base + k * 16, 16)]
                dx = xs - cxj
                dy = ys - cyj
                dz = zs - czj
                d2 = (dx * dx + dz * dz) + dy * dy
                ms.append(d2 <= _R2)
                pcs.append(plsc.all_reduce_population_count(ms[k]))
            offs = [cnt]
            s = pcs[0]
            for k in range(1, 8):
                offs.append(cnt + s[0])
                s = s + pcs[k]
            for k in range(8):
                plsc.store_compressed(selv.at[pl.ds(offs[k], 16)],
                                      iota + (base + k * 16), mask=ms[k])
            return base + 128, cnt + s[0]

        _, cnt = lax.while_loop(cond, body, (jnp.int32(0), jnp.int32(0)))
        cnt = jnp.minimum(cnt, _K)
        first = jnp.where(cnt > 0, selv[pl.ds(0, 16)][0], 0)
        tsplat = jnp.full((16,), t, jnp.int32)
        for k in range(_K // 16):
            lane = iota + (k * 16)
            sel = selv[pl.ds(k * 16, 16)]
            sel = jnp.where(lane < cnt, sel, first)
            gx = plsc.load_gather(xv, [sel]) - cxj
            gy = plsc.load_gather(yv, [sel]) - cyj
            gz = plsc.load_gather(zv, [sel]) - czj
            pos0 = lane * 3
            plsc.store_scatter(outv, [tsplat, pos0], gx)
            plsc.store_scatter(outv, [tsplat, pos0 + 1], gy)
            plsc.store_scatter(outv, [tsplat, pos0 + 2], gz)
        return carry

    lax.fori_loop(0, _RPW, row, jnp.int32(0))
    pltpu.async_copy(outv, out.at[idxv], sem).wait()


@functools.cache
def _build_ballq():
    return functools.partial(
        pl.kernel,
        out_type=jax.ShapeDtypeStruct((_B * _S, 256), jnp.float32),
        mesh=plsc.VectorSubcoreMesh(core_axis_name="c", subcore_axis_name="s"),
        compiler_params=pltpu.CompilerParams(needs_layout_passes=False),
        scratch_types=[
            pltpu.VMEM((_N,), jnp.float32),
            pltpu.VMEM((_N,), jnp.float32),
            pltpu.VMEM((_N,), jnp.float32),
            pltpu.VMEM((_S,), jnp.float32),
            pltpu.VMEM((_S,), jnp.float32),
            pltpu.VMEM((_S,), jnp.float32),
            pltpu.VMEM((192,), jnp.int32),
            pltpu.VMEM((_RPW, 256), jnp.float32),
            pltpu.VMEM((_RPW,), jnp.int32),
            pltpu.SemaphoreType.DMA,
        ],
    )(_ballq_body)


def kernel(xyz):
    xp = xyz[:, :, 0]
    yp = xyz[:, :, 1]
    zp = xyz[:, :, 2]

    cent16 = pl.pallas_call(
        _fps_body,
        out_shape=jax.ShapeDtypeStruct((_S, 16), jnp.float32),
        scratch_shapes=[pltpu.VMEM((_B, _ROWS, _LANES), jnp.float32)],
    )(xp.reshape(_B, _ROWS, _LANES),
      yp.reshape(_B, _ROWS, _LANES),
      zp.reshape(_B, _ROWS, _LANES))

    centroids = cent16.reshape(_S, 4, 4)[:, :, :3].transpose(1, 0, 2)

    flat = _build_ballq()(xp, yp, zp,
                          centroids[:, :, 0], centroids[:, :, 1],
                          centroids[:, :, 2])
    new_xyz = flat[:, :_K * 3].reshape(_B, _S, _K, 3)
    return new_xyz, centroids


# speedup vs baseline: 1.8870x; 1.0652x over previous
import functools

import jax
import jax.numpy as jnp
from jax import lax
from jax.experimental import pallas as pl
from jax.experimental.pallas import tpu as pltpu
from jax.experimental.pallas import tpu_sc as plsc

_N = 8192
_S = 1024
_K = 64
_B = 4
_R2 = 0.2 * 0.2
_ROWS = 64
_LANES = 128

_NW = 32
_RPW = (_B * _S) // _NW


def _fps_body(xp_ref, yp_ref, zp_ref, out_ref, dist_ref):
    iota3 = (lax.broadcasted_iota(jnp.int32, (_B, _ROWS, _LANES), 1) * _LANES
             + lax.broadcasted_iota(jnp.int32, (_B, _ROWS, _LANES), 2))
    dist_ref[...] = jnp.full((_B, _ROWS, _LANES), 1e10, jnp.float32)
    c0x = xp_ref[:, 0:1, 0:1]
    c0y = yp_ref[:, 0:1, 0:1]
    c0z = zp_ref[:, 0:1, 0:1]

    def half(i_out, cx, cy, cz, dprev, xs, ys, zs):
        for b in range(_B):
            out_ref[pl.ds(i_out, 1), pl.ds(b * 4 + 0, 1)] = cx[b].reshape(1, 1)
            out_ref[pl.ds(i_out, 1), pl.ds(b * 4 + 1, 1)] = cy[b].reshape(1, 1)
            out_ref[pl.ds(i_out, 1), pl.ds(b * 4 + 2, 1)] = cz[b].reshape(1, 1)
        dx = xs - cx
        dy = ys - cy
        dz = zs - cz
        d = (dx * dx + dz * dz) + dy * dy
        dmin = jnp.minimum(dprev, d)
        mx = jnp.max(dmin, axis=(1, 2), keepdims=True)
        eqm = dmin == mx
        ncx = jnp.sum(jnp.where(eqm, xs, 0.0), axis=(1, 2), keepdims=True)
        ncy = jnp.sum(jnp.where(eqm, ys, 0.0), axis=(1, 2), keepdims=True)
        ncz = jnp.sum(jnp.where(eqm, zs, 0.0), axis=(1, 2), keepdims=True)
        nmax = jnp.sum(jnp.where(eqm, 1, 0), axis=(1, 2), keepdims=True)

        def tie_path(_):
            nf = jnp.min(jnp.where(eqm, iota3, _N), axis=(1, 2), keepdims=True)
            pm = iota3 == nf
            tx = jnp.sum(jnp.where(pm, xs, 0.0), axis=(1, 2), keepdims=True)
            ty = jnp.sum(jnp.where(pm, ys, 0.0), axis=(1, 2), keepdims=True)
            tz = jnp.sum(jnp.where(pm, zs, 0.0), axis=(1, 2), keepdims=True)
            return tx, ty, tz

        c = lax.cond(jnp.max(nmax) > 1, tie_path,
                     lambda _: (ncx, ncy, ncz), 0)
        return c[0], c[1], c[2], dmin

    def step(i, carry):
        cx, cy, cz = carry
        xs = xp_ref[...]
        ys = yp_ref[...]
        zs = zp_ref[...]
        dmin = dist_ref[...]
        for u in range(4):
            cx, cy, cz, dmin = half(4 * i + u, cx, cy, cz, dmin, xs, ys, zs)
        dist_ref[...] = dmin
        return cx, cy, cz

    lax.fori_loop(0, _S // 4, step, (c0x, c0y, c0z))


def _ballq_body(xp, yp, zp, cxp, cyp, czp, out,
                xv, yv, zv, cxv, cyv, czv, selv, outv, idxv, sem):
    cidx = lax.axis_index("c")
    sidx = lax.axis_index("s")
    wid = sidx * 2 + cidx
    b = wid // (_NW // _B)
    sub = wid % (_NW // _B)

    pltpu.sync_copy(xp.at[b], xv)
    pltpu.sync_copy(yp.at[b], yv)
    pltpu.sync_copy(zp.at[b], zv)
    pltpu.sync_copy(cxp.at[b], cxv)
    pltpu.sync_copy(cyp.at[b], cyv)
    pltpu.sync_copy(czp.at[b], czv)

    iota = lax.iota(jnp.int32, 16)
    nb = _NW // _B

    for t16 in range(_RPW // 16):
        idxv[pl.ds(t16 * 16, 16)] = (iota + t16 * 16) * nb + sub + b * _S

    def row(t, carry):
        s = t * nb + sub
        ssplat = jnp.full((16,), s, jnp.int32)
        cxj = plsc.load_gather(cxv, [ssplat])
        cyj = plsc.load_gather(cyv, [ssplat])
        czj = plsc.load_gather(czv, [ssplat])

        def cond(st):
            base, cnt = st
            return jnp.logical_and(cnt < _K, base < _N)

        def body(st):
            base, cnt = st
            ms = []
            pcs = []
            for k in range(16):
                xs = xv[pl.ds(base + k * 16, 16)]
                ys = yv[pl.ds(base + k * 16, 16)]
                zs = zv[pl.ds(base + k * 16, 16)]
                dx = xs - cxj
                dy = ys - cyj
                dz = zs - czj
                d2 = (dx * dx + dz * dz) + dy * dy
                ms.append(d2 <= _R2)
                pcs.append(plsc.all_reduce_population_count(ms[k]))
            offs = [cnt]
            s = pcs[0]
            for k in range(1, 16):
                offs.append(cnt + s[0])
                s = s + pcs[k]
            for k in range(16):
                plsc.store_compressed(selv.at[pl.ds(offs[k], 16)],
                                      iota + (base + k * 16), mask=ms[k])
            return base + 256, cnt + s[0]

        _, cnt = lax.while_loop(cond, body, (jnp.int32(0), jnp.int32(0)))
        cnt = jnp.minimum(cnt, _K)
        first = jnp.where(cnt > 0, selv[pl.ds(0, 16)][0], 0)
        tsplat = jnp.full((16,), t, jnp.int32)
        for k in range(_K // 16):
            lane = iota + (k * 16)
            sel = selv[pl.ds(k * 16, 16)]
            sel = jnp.where(lane < cnt, sel, first)
            gx = plsc.load_gather(xv, [sel]) - cxj
            gy = plsc.load_gather(yv, [sel]) - cyj
            gz = plsc.load_gather(zv, [sel]) - czj
            pos0 = lane * 3
            plsc.store_scatter(outv, [tsplat, pos0], gx)
            plsc.store_scatter(outv, [tsplat, pos0 + 1], gy)
            plsc.store_scatter(outv, [tsplat, pos0 + 2], gz)
        return carry

    lax.fori_loop(0, _RPW, row, jnp.int32(0))
    pltpu.async_copy(outv, out.at[idxv], sem).wait()


@functools.cache
def _build_ballq():
    return functools.partial(
        pl.kernel,
        out_type=jax.ShapeDtypeStruct((_B * _S, 256), jnp.float32),
        mesh=plsc.VectorSubcoreMesh(core_axis_name="c", subcore_axis_name="s"),
        compiler_params=pltpu.CompilerParams(needs_layout_passes=False),
        scratch_types=[
            pltpu.VMEM((_N,), jnp.float32),
            pltpu.VMEM((_N,), jnp.float32),
            pltpu.VMEM((_N,), jnp.float32),
            pltpu.VMEM((_S,), jnp.float32),
            pltpu.VMEM((_S,), jnp.float32),
            pltpu.VMEM((_S,), jnp.float32),
            pltpu.VMEM((336,), jnp.int32),
            pltpu.VMEM((_RPW, 256), jnp.float32),
            pltpu.VMEM((_RPW,), jnp.int32),
            pltpu.SemaphoreType.DMA,
        ],
    )(_ballq_body)


def kernel(xyz):
    xp = xyz[:, :, 0]
    yp = xyz[:, :, 1]
    zp = xyz[:, :, 2]

    cent16 = pl.pallas_call(
        _fps_body,
        out_shape=jax.ShapeDtypeStruct((_S, 16), jnp.float32),
        scratch_shapes=[pltpu.VMEM((_B, _ROWS, _LANES), jnp.float32)],
    )(xp.reshape(_B, _ROWS, _LANES),
      yp.reshape(_B, _ROWS, _LANES),
      zp.reshape(_B, _ROWS, _LANES))

    centroids = cent16.reshape(_S, 4, 4)[:, :, :3].transpose(1, 0, 2)

    flat = _build_ballq()(xp, yp, zp,
                          centroids[:, :, 0], centroids[:, :, 1],
                          centroids[:, :, 2])
    new_xyz = flat[:, :_K * 3].reshape(_B, _S, _K, 3)
    return new_xyz, centroids
